# Initial kernel scaffold; baseline (speedup 1.0000x reference)
import functools

import jax
import jax.numpy as jnp
from jax import lax
from jax.experimental import pallas as pl
from jax.experimental.pallas import tpu as pltpu
from jax.experimental.pallas import tpu_sc as plsc

_NC = 2
_NS = 16
_NW = _NC * _NS
_L = 16

_CE = 4
_NBUF = 4
_PAD = 40.0


def _make_sc_scores(B, R, C, D, V):
    ew = B // _NW
    n_chunks = ew // _CE
    rpc = R * _CE
    mesh = plsc.VectorSubcoreMesh(
        core_axis_name="c", subcore_axis_name="s",
        num_cores=_NC, num_subcores=_NS)

    @functools.partial(
        pl.kernel,
        out_type=jax.ShapeDtypeStruct((B, 32), jnp.float32),
        mesh=mesh,
        scratch_types=[
            pltpu.VMEM((n_chunks, rpc), jnp.int32),
            pltpu.VMEM((_NBUF, rpc, D), jnp.float32),
            pltpu.VMEM((ew, 32), jnp.float32),
        ] + [pltpu.SemaphoreType.DMA] * _NBUF,
    )
    def sc_scores(idx_hbm, emb_hbm, out_hbm, idx_v, rows_v, scores_v, *sems):
        wid = lax.axis_index("s") * _NC + lax.axis_index("c")
        pltpu.sync_copy(idx_hbm.at[wid], idx_v)

        lane = lax.iota(jnp.int32, (16,))
        pad = jnp.full((_L,), _PAD, jnp.float32)

        def fire(c, b):
            pltpu.make_async_copy(
                emb_hbm.at[idx_v.at[c]], rows_v.at[b], sems[b]).start()

        for b in range(_NBUF):
            fire(b, b)

        @pl.loop(0, n_chunks, step=_NBUF)
        def _chunks(c0):
            for b in range(_NBUF):
                c = c0 + b
                pltpu.make_async_copy(
                    emb_hbm.at[idx_v.at[c]], rows_v.at[b], sems[b]).wait()
                for kk in range(_CE):
                    base = kk * R
                    tl = rows_v[b, base, pl.ds(0, _L)]
                    th = rows_v[b, base, pl.ds(_L, _L)]
                    cl = rows_v[b, base + 1, pl.ds(0, _L)]
                    ch = rows_v[b, base + 1, pl.ds(_L, _L)]
                    for i in range(2, C + 1):
                        cl = cl + rows_v[b, base + i, pl.ds(0, _L)]
                        ch = ch + rows_v[b, base + i, pl.ds(_L, _L)]
                    ncl = -cl
                    nch = -ch

                    def dot(rl, rh, sl, sh):
                        s = jnp.sum(rl * sl + rh * sh)
                        return lax.broadcast(s, (_L,))

                    acc0 = jnp.where(lane == 0, dot(tl, th, cl, ch), pad)
                    acc1 = pad
                    for j in range(R - C - 1):
                        rl = rows_v[b, base + C + 1 + j, pl.ds(0, _L)]
                        rh = rows_v[b, base + C + 1 + j, pl.ds(_L, _L)]
                        sv = dot(rl, rh, ncl, nch)
                        t = 1 + j
                        if t < _L:
                            acc0 = jnp.where(lane == t, sv, acc0)
                        else:
                            acc1 = jnp.where(lane == t - _L, sv, acc1)
                    e_loc = c * _CE + kk
                    scores_v[e_loc, pl.ds(0, _L)] = acc0
                    scores_v[e_loc, pl.ds(_L, _L)] = acc1
                pl.when(c + _NBUF < n_chunks)(lambda: fire(c + _NBUF, b))

        pltpu.sync_copy(scores_v, out_hbm.at[pl.ds(wid * ew, ew)])

    return sc_scores


def _tc_loss(scores):
    def body(x_ref, o_ref):
        o_ref[0, 0] = -jnp.sum(jax.nn.log_sigmoid(x_ref[...]))

    return pl.pallas_call(
        body,
        out_shape=jax.ShapeDtypeStruct((1, 1), jnp.float32),
    )(scores)


def kernel(context_words, target_word, negative_words, embeddings):
    B, C = context_words.shape
    K = negative_words.shape[1]
    V, D = embeddings.shape
    R = 1 + C + K
    assert D == 2 * _L and B % (_NW * _CE) == 0

    idx = jnp.concatenate(
        [target_word.astype(jnp.int32),
         context_words.astype(jnp.int32),
         negative_words.astype(jnp.int32)], axis=1)
    idx3 = idx.reshape(_NW, (B // _NW) // _CE, R * _CE)

    scores = _make_sc_scores(B, R, C, D, V)(idx3, embeddings)
    loss = _tc_loss(scores.reshape(B * 32 // 128, 128))
    return loss[0, 0]


# speedup vs baseline: 2.1810x; 2.1810x over previous
import functools

import jax
import jax.numpy as jnp
import numpy as np
from jax import lax
from jax.experimental import pallas as pl
from jax.experimental.pallas import tpu as pltpu
from jax.experimental.pallas import tpu_sc as plsc

_NC = 2
_NS = 16
_NW = _NC * _NS
_L = 16

_CE = 4
_NBUF = 4
_PAD = 40.0


def _shuf(x, perm):
    return lax.gather(
        x, perm,
        lax.GatherDimensionNumbers(
            offset_dims=(), collapsed_slice_dims=(0,), start_index_map=(0,)),
        slice_sizes=(1,),
        unique_indices=True, indices_are_sorted=False,
        mode=lax.GatherScatterMode.PROMISE_IN_BOUNDS)


def _lane_sums(vecs, perms, masks):
    assert len(vecs) == _L
    for s in range(4):
        perm, mask = perms[s], masks[s]
        nxt = []
        for m in range(0, len(vecs), 2):
            a, b = vecs[m], vecs[m + 1]
            fa = 2.0 * a if isinstance(a, float) else a + _shuf(a, perm)
            fb = 2.0 * b if isinstance(b, float) else b + _shuf(b, perm)
            if isinstance(fa, float) and isinstance(fb, float):
                nxt.append(fa if fa == fb else
                           jnp.where(mask, jnp.full((_L,), fa, jnp.float32),
                                     jnp.full((_L,), fb, jnp.float32)))
            else:
                if isinstance(fa, float):
                    fa = jnp.full((_L,), fa, jnp.float32)
                if isinstance(fb, float):
                    fb = jnp.full((_L,), fb, jnp.float32)
                nxt.append(jnp.where(mask, fa, fb))
        vecs = nxt
    return vecs[0]


def _make_sc_scores(B, R, C, D, V):
    ew = B // _NW
    n_chunks = ew // _CE
    rpc = R * _CE
    mesh = plsc.VectorSubcoreMesh(
        core_axis_name="c", subcore_axis_name="s",
        num_cores=_NC, num_subcores=_NS)

    @functools.partial(
        pl.kernel,
        out_type=jax.ShapeDtypeStruct((B, 32), jnp.float32),
        mesh=mesh,
        scratch_types=[
            pltpu.VMEM((n_chunks, rpc), jnp.int32),
            pltpu.VMEM((_NBUF, rpc, D), jnp.float32),
            pltpu.VMEM((ew, 32), jnp.float32),
        ] + [pltpu.SemaphoreType.DMA] * _NBUF,
        compiler_params=pltpu.CompilerParams(use_tc_tiling_on_sc=False),
    )
    def sc_scores(idx_hbm, emb_hbm, out_hbm, idx_v, rows_v, scores_v, *sems):
        wid = lax.axis_index("s") * _NC + lax.axis_index("c")
        pltpu.sync_copy(idx_hbm.at[wid], idx_v)

        lane = lax.iota(jnp.int32, _L)
        perms = [jnp.reshape(lane ^ (1 << s), (_L, 1)) for s in range(4)]
        masks = [(lane & (1 << s)) == 0 for s in range(4)]

        def fire(c, b):
            pltpu.make_async_copy(
                emb_hbm.at[idx_v.at[c]], rows_v.at[b], sems[b]).start()

        for b in range(_NBUF):
            fire(b, b)

        @pl.loop(0, n_chunks, step=_NBUF)
        def _chunks(c0):
            for b in range(_NBUF):
                c = c0 + b
                pltpu.make_async_copy(
                    emb_hbm.at[idx_v.at[c]], rows_v.at[b], sems[b]).wait()
                for kk in range(_CE):
                    base = kk * R
                    tl = rows_v[b, base, pl.ds(0, _L)]
                    th = rows_v[b, base, pl.ds(_L, _L)]
                    cl = rows_v[b, base + 1, pl.ds(0, _L)]
                    ch = rows_v[b, base + 1, pl.ds(_L, _L)]
                    for i in range(2, C + 1):
                        cl = cl + rows_v[b, base + i, pl.ds(0, _L)]
                        ch = ch + rows_v[b, base + i, pl.ds(_L, _L)]
                    ncl = -cl
                    nch = -ch

                    ps = [tl * cl + th * ch]
                    for j in range(R - C - 1):
                        rl = rows_v[b, base + C + 1 + j, pl.ds(0, _L)]
                        rh = rows_v[b, base + C + 1 + j, pl.ds(_L, _L)]
                        ps.append(rl * ncl + rh * nch)
                    ps += [_PAD / _L] * (2 * _L - len(ps))
                    e_loc = c * _CE + kk
                    scores_v[e_loc, pl.ds(0, _L)] = _lane_sums(
                        ps[:_L], perms, masks)
                    scores_v[e_loc, pl.ds(_L, _L)] = _lane_sums(
                        ps[_L:], perms, masks)
                pl.when(c + _NBUF < n_chunks)(lambda: fire(c + _NBUF, b))

        pltpu.sync_copy(scores_v, out_hbm.at[pl.ds(wid * ew, ew)])

    return sc_scores


def _tc_loss(scores):
    def body(x_ref, o_ref):
        o_ref[...] = (-jnp.sum(jax.nn.log_sigmoid(x_ref[...]))).reshape(1, 1)

    return pl.pallas_call(
        body,
        out_shape=jax.ShapeDtypeStruct((1, 1), jnp.float32),
    )(scores)


def kernel(context_words, target_word, negative_words, embeddings):
    B, C = context_words.shape
    K = negative_words.shape[1]
    V, D = embeddings.shape
    R = 1 + C + K
    assert D == 2 * _L and B % (_NW * _CE) == 0

    idx = jnp.concatenate(
        [target_word.astype(jnp.int32),
         context_words.astype(jnp.int32),
         negative_words.astype(jnp.int32)], axis=1)
    idx3 = idx.reshape(_NW, (B // _NW) // _CE, R * _CE)

    scores = _make_sc_scores(B, R, C, D, V)(idx3, embeddings)
    loss = _tc_loss(scores.reshape(B * 32 // 128, 128))
    return loss[0, 0]


# speedup vs baseline: 2.4591x; 1.1275x over previous
import functools

import jax
import jax.numpy as jnp
import numpy as np
from jax import lax
from jax.experimental import pallas as pl
from jax.experimental.pallas import tpu as pltpu
from jax.experimental.pallas import tpu_sc as plsc

_NC = 2
_NS = 16
_NW = _NC * _NS
_L = 16

_CE = 4
_NBUF = 4
_PAD = 40.0


def _shuf(x, perm):
    return lax.gather(
        x, perm,
        lax.GatherDimensionNumbers(
            offset_dims=(), collapsed_slice_dims=(0,), start_index_map=(0,)),
        slice_sizes=(1,),
        unique_indices=True, indices_are_sorted=False,
        mode=lax.GatherScatterMode.PROMISE_IN_BOUNDS)


def _lane_sums(vecs, perms, masks):
    assert len(vecs) == _L
    for s in range(4):
        perm, mask = perms[s], masks[s]
        nxt = []
        for m in range(0, len(vecs), 2):
            a, b = vecs[m], vecs[m + 1]
            fa = 2.0 * a if isinstance(a, float) else a + _shuf(a, perm)
            fb = 2.0 * b if isinstance(b, float) else b + _shuf(b, perm)
            if isinstance(fa, float) and isinstance(fb, float):
                nxt.append(fa if fa == fb else
                           jnp.where(mask, jnp.full((_L,), fa, jnp.float32),
                                     jnp.full((_L,), fb, jnp.float32)))
            else:
                if isinstance(fa, float):
                    fa = jnp.full((_L,), fa, jnp.float32)
                if isinstance(fb, float):
                    fb = jnp.full((_L,), fb, jnp.float32)
                nxt.append(jnp.where(mask, fa, fb))
        vecs = nxt
    return vecs[0]


def _make_sc_scores(B, R, C, D, V):
    ew = B // _NW
    n_chunks = ew // _CE
    rpc = R * _CE
    mesh = plsc.VectorSubcoreMesh(
        core_axis_name="c", subcore_axis_name="s",
        num_cores=_NC, num_subcores=_NS)

    @functools.partial(
        pl.kernel,
        out_type=jax.ShapeDtypeStruct((B, 32), jnp.float32),
        mesh=mesh,
        scratch_types=[
            pltpu.VMEM((n_chunks, rpc), jnp.int32),
            pltpu.VMEM((_NBUF, rpc, D), jnp.float32),
            pltpu.VMEM((ew, 32), jnp.float32),
        ] + [pltpu.SemaphoreType.DMA] * _NBUF,
        compiler_params=pltpu.CompilerParams(use_tc_tiling_on_sc=False),
    )
    def sc_scores(idx_hbm, emb_hbm, out_hbm, idx_v, rows_v, scores_v, *sems):
        wid = lax.axis_index("s") * _NC + lax.axis_index("c")
        pltpu.sync_copy(idx_hbm.at[wid], idx_v)

        lane = lax.iota(jnp.int32, _L)
        perms = [jnp.reshape(lane ^ (1 << s), (_L, 1)) for s in range(4)]
        masks = [(lane & (1 << s)) == 0 for s in range(4)]

        def fire(c, b):
            pltpu.make_async_copy(
                emb_hbm.at[idx_v.at[c]], rows_v.at[b], sems[b]).start()

        for b in range(_NBUF):
            fire(b, b)

        @pl.loop(0, n_chunks, step=_NBUF)
        def _chunks(c0):
            for b in range(_NBUF):
                c = c0 + b
                pltpu.make_async_copy(
                    emb_hbm.at[idx_v.at[c]], rows_v.at[b], sems[b]).wait()
                for kk in range(_CE):
                    base = kk * R
                    tl = rows_v[b, base, pl.ds(0, _L)]
                    th = rows_v[b, base, pl.ds(_L, _L)]
                    cl = rows_v[b, base + 1, pl.ds(0, _L)]
                    ch = rows_v[b, base + 1, pl.ds(_L, _L)]
                    for i in range(2, C + 1):
                        cl = cl + rows_v[b, base + i, pl.ds(0, _L)]
                        ch = ch + rows_v[b, base + i, pl.ds(_L, _L)]
                    ncl = -cl
                    nch = -ch

                    ps = [tl * cl + th * ch]
                    for j in range(R - C - 1):
                        rl = rows_v[b, base + C + 1 + j, pl.ds(0, _L)]
                        rh = rows_v[b, base + C + 1 + j, pl.ds(_L, _L)]
                        ps.append(rl * ncl + rh * nch)
                    ps += [_PAD / _L] * (2 * _L - len(ps))
                    e_loc = c * _CE + kk
                    scores_v[e_loc, pl.ds(0, _L)] = _lane_sums(
                        ps[:_L], perms, masks)
                    scores_v[e_loc, pl.ds(_L, _L)] = _lane_sums(
                        ps[_L:], perms, masks)
                pl.when(c + _NBUF < n_chunks)(lambda: fire(c + _NBUF, b))

        pltpu.sync_copy(scores_v, out_hbm.at[pl.ds(wid * ew, ew)])

    return sc_scores


_RET_B = 512


def _tc_retile(embT, V):
    blkc = 4 * _RET_B
    grid = (V + blkc - 1) // blkc

    def body(x_ref, o_ref):
        x = x_ref[...]
        o_ref[...] = jnp.concatenate(
            [jnp.transpose(x[:, q * _RET_B:(q + 1) * _RET_B], (1, 0))
             for q in range(4)], axis=1)

    return pl.pallas_call(
        body,
        grid=(grid,),
        in_specs=[pl.BlockSpec((32, blkc), lambda c: (0, c))],
        out_specs=pl.BlockSpec((_RET_B, 128), lambda c: (c, 0)),
        out_shape=jax.ShapeDtypeStruct((grid * _RET_B, 128), jnp.float32),
    )(embT)


def _remap_idx(i):
    blkc = 4 * _RET_B
    return 4 * (_RET_B * (i // blkc) + (i % _RET_B)) + (i % blkc) // _RET_B


def _tc_loss(scores):
    def body(x_ref, o_ref):
        o_ref[...] = (-jnp.sum(jax.nn.log_sigmoid(x_ref[...]))).reshape(1, 1)

    return pl.pallas_call(
        body,
        out_shape=jax.ShapeDtypeStruct((1, 1), jnp.float32),
    )(scores)


def kernel(context_words, target_word, negative_words, embeddings):
    B, C = context_words.shape
    K = negative_words.shape[1]
    V, D = embeddings.shape
    R = 1 + C + K
    assert D == 2 * _L and B % (_NW * _CE) == 0

    idx = jnp.concatenate(
        [target_word.astype(jnp.int32),
         context_words.astype(jnp.int32),
         negative_words.astype(jnp.int32)], axis=1)
    idx = _remap_idx(idx)
    idx3 = idx.reshape(_NW, (B // _NW) // _CE, R * _CE)

    t128 = _tc_retile(embeddings.T, V)
    emb_rm = t128.reshape(4 * t128.shape[0], D)
    scores = _make_sc_scores(B, R, C, D, emb_rm.shape[0])(idx3, emb_rm)
    loss = _tc_loss(scores.reshape(B * 32 // 128, 128))
    return loss[0, 0]
